# Initial kernel scaffold; baseline (speedup 1.0000x reference)
import jax
import jax.numpy as jnp
from jax.experimental import pallas as pl


def kernel(x, edge_index, W_msg, b_msg, W_upd, b_upd):
    raise NotImplementedError("write your pallas kernel here")


# speedup vs baseline: 3.2018x; 3.2018x over previous
import functools

import jax
import jax.numpy as jnp
from jax import lax
from jax.experimental import pallas as pl
from jax.experimental.pallas import tpu as pltpu
from jax.experimental.pallas import tpu_sc as plsc

N_NODES = 10000
D = 128
N_EDGES = 320000
NUM_CORES = 2
NUM_SUBCORES = 16
NW = NUM_CORES * NUM_SUBCORES
K = 128
NCH = 80
EPW = NCH * K
E_PAD = NW * EPW
N_ACC = 10240
RPS = N_ACC // NUM_SUBCORES
BLK = 1024


def _msg_matmul(x, wt, b):
  def body(x_ref, w_ref, b_ref, o_ref):
    o_ref[...] = (
        jnp.dot(x_ref[...], w_ref[...], preferred_element_type=jnp.float32)
        + b_ref[...]
    )
  return pl.pallas_call(
      body,
      grid=(pl.cdiv(N_NODES, BLK),),
      in_specs=[
          pl.BlockSpec((BLK, D), lambda j: (j, 0)),
          pl.BlockSpec((D, D), lambda j: (0, 0)),
          pl.BlockSpec((1, D), lambda j: (0, 0)),
      ],
      out_specs=pl.BlockSpec((BLK, D), lambda j: (j, 0)),
      out_shape=jax.ShapeDtypeStruct((N_NODES, D), jnp.float32),
  )(x, wt, b)


def _update_matmul(partials, wt, b):
  def body(p_ref, w_ref, b_ref, o_ref):
    s = p_ref[0] + p_ref[1]
    o_ref[...] = (
        jnp.dot(s, w_ref[...], preferred_element_type=jnp.float32) + b_ref[...]
    )
  return pl.pallas_call(
      body,
      grid=(pl.cdiv(N_NODES, BLK),),
      in_specs=[
          pl.BlockSpec((NUM_CORES, BLK, D), lambda j: (0, j, 0)),
          pl.BlockSpec((D, D), lambda j: (0, 0)),
          pl.BlockSpec((1, D), lambda j: (0, 0)),
      ],
      out_specs=pl.BlockSpec((BLK, D), lambda j: (j, 0)),
      out_shape=jax.ShapeDtypeStruct((N_NODES, D), jnp.float32),
  )(partials, wt, b)


@functools.cache
def _make_sc_gather_scatter_add():
  @functools.partial(
      pl.kernel,
      mesh=plsc.VectorSubcoreMesh(core_axis_name="c", subcore_axis_name="s"),
      out_type=jax.ShapeDtypeStruct((NUM_CORES, N_ACC, D), jnp.float32),
      scratch_types=[
          pltpu.VMEM((NCH, K), jnp.int32),
          pltpu.VMEM((NCH, K), jnp.int32),
          pltpu.VMEM((K, D), jnp.float32),
          pltpu.VMEM_SHARED((N_ACC, D), jnp.float32),
          pltpu.SemaphoreType.DMA,
      ],
  )
  def _sc_gather_scatter_add(y_hbm, src_hbm, dst_hbm, zero_hbm, out_hbm,
                             src_v, dst_v, gbuf, agg_sh, gsem):
    cid = lax.axis_index("c")
    sid = lax.axis_index("s")
    wid = sid * NUM_CORES + cid
    base = sid * RPS

    pltpu.sync_copy(zero_hbm, agg_sh.at[pl.ds(base, RPS)])
    pltpu.sync_copy(src_hbm.at[wid], src_v)
    pltpu.sync_copy(dst_hbm.at[wid], dst_v)
    plsc.subcore_barrier()

    def chunk(j, carry):
      pltpu.async_copy(y_hbm.at[src_v.at[j]], gbuf, gsem).wait()
      pltpu.sync_copy(gbuf, agg_sh.at[dst_v.at[j]], add=True)
      return carry

    lax.fori_loop(0, NCH, chunk, 0)

    plsc.subcore_barrier()
    pltpu.sync_copy(agg_sh.at[pl.ds(base, RPS)],
                    out_hbm.at[cid].at[pl.ds(base, RPS)])

  return _sc_gather_scatter_add


def kernel(x, edge_index, W_msg, b_msg, W_upd, b_upd):
  src = edge_index[0].astype(jnp.int32)
  dst = edge_index[1].astype(jnp.int32)
  pad = E_PAD - N_EDGES
  src_p = jnp.concatenate([src, jnp.zeros((pad,), jnp.int32)]).reshape(NW, NCH, K)
  dst_p = jnp.concatenate([dst, jnp.full((pad,), N_NODES, jnp.int32)]).reshape(NW, NCH, K)
  zero = jnp.zeros((RPS, D), jnp.float32)

  y = _msg_matmul(x, W_msg.T, b_msg.reshape(1, D))
  partials = _make_sc_gather_scatter_add()(y, src_p, dst_p, zero)
  return _update_matmul(partials, W_upd.T, b_upd.reshape(1, D))


# speedup vs baseline: 3.2820x; 1.0250x over previous
import functools

import jax
import jax.numpy as jnp
from jax import lax
from jax.experimental import pallas as pl
from jax.experimental.pallas import tpu as pltpu
from jax.experimental.pallas import tpu_sc as plsc

N_NODES = 10000
D = 128
N_EDGES = 320000
NUM_CORES = 2
NUM_SUBCORES = 16
NW = NUM_CORES * NUM_SUBCORES
K = 128
NCH = 80
NCH2 = NCH // 2
EPW = NCH * K
E_PAD = NW * EPW
N_ACC = 10240
RPS = N_ACC // NUM_SUBCORES
BLK = 1024


def _msg_matmul(x, wt, b):
  def body(x_ref, w_ref, b_ref, o_ref):
    o_ref[...] = (
        jnp.dot(x_ref[...], w_ref[...], preferred_element_type=jnp.float32)
        + b_ref[...]
    )
  return pl.pallas_call(
      body,
      grid=(pl.cdiv(N_NODES, BLK),),
      in_specs=[
          pl.BlockSpec((BLK, D), lambda j: (j, 0)),
          pl.BlockSpec((D, D), lambda j: (0, 0)),
          pl.BlockSpec((1, D), lambda j: (0, 0)),
      ],
      out_specs=pl.BlockSpec((BLK, D), lambda j: (j, 0)),
      out_shape=jax.ShapeDtypeStruct((N_NODES, D), jnp.float32),
  )(x, wt, b)


def _update_matmul(partials, wt, b):
  def body(p_ref, w_ref, b_ref, o_ref):
    s = p_ref[0] + p_ref[1]
    o_ref[...] = (
        jnp.dot(s, w_ref[...], preferred_element_type=jnp.float32) + b_ref[...]
    )
  return pl.pallas_call(
      body,
      grid=(pl.cdiv(N_NODES, BLK),),
      in_specs=[
          pl.BlockSpec((NUM_CORES, BLK, D), lambda j: (0, j, 0)),
          pl.BlockSpec((D, D), lambda j: (0, 0)),
          pl.BlockSpec((1, D), lambda j: (0, 0)),
      ],
      out_specs=pl.BlockSpec((BLK, D), lambda j: (j, 0)),
      out_shape=jax.ShapeDtypeStruct((N_NODES, D), jnp.float32),
  )(partials, wt, b)


@functools.cache
def _make_sc_gather_scatter_add():
  @functools.partial(
      pl.kernel,
      mesh=plsc.VectorSubcoreMesh(core_axis_name="c", subcore_axis_name="s"),
      out_type=jax.ShapeDtypeStruct((NUM_CORES, N_ACC, D), jnp.float32),
      scratch_types=[
          pltpu.VMEM((NCH2, K), jnp.int32),
          pltpu.VMEM((NCH2, K), jnp.int32),
          pltpu.VMEM((2, K, D), jnp.float32),
          pltpu.VMEM_SHARED((N_ACC, D), jnp.float32),
          pltpu.SemaphoreType.DMA,
          pltpu.SemaphoreType.DMA,
          pltpu.SemaphoreType.DMA,
          pltpu.SemaphoreType.DMA,
      ],
  )
  def _sc_gather_scatter_add(y_hbm, src_hbm, dst_hbm, zero_hbm, out_hbm,
                             src_v, dst_v, gbuf, agg_sh,
                             gsem0, gsem1, ssem0, ssem1):
    cid = lax.axis_index("c")
    sid = lax.axis_index("s")
    wid = sid * NUM_CORES + cid
    base = sid * RPS

    pltpu.sync_copy(zero_hbm, agg_sh.at[pl.ds(base, RPS)])
    plsc.subcore_barrier()

    for h in range(2):
      pltpu.sync_copy(src_hbm.at[wid].at[pl.ds(h * NCH2, NCH2)], src_v)
      pltpu.sync_copy(dst_hbm.at[wid].at[pl.ds(h * NCH2, NCH2)], dst_v)

      def pair(i, carry):
        j0 = 2 * i
        j1 = 2 * i + 1
        g0 = pltpu.async_copy(y_hbm.at[src_v.at[j0]], gbuf.at[0], gsem0)
        g1 = pltpu.async_copy(y_hbm.at[src_v.at[j1]], gbuf.at[1], gsem1)
        g0.wait()
        s0 = pltpu.async_copy(gbuf.at[0], agg_sh.at[dst_v.at[j0]], ssem0,
                              add=True)
        g1.wait()
        s1 = pltpu.async_copy(gbuf.at[1], agg_sh.at[dst_v.at[j1]], ssem1,
                              add=True)
        s0.wait()
        s1.wait()
        return carry

      lax.fori_loop(0, NCH2 // 2, pair, 0)

    plsc.subcore_barrier()
    pltpu.sync_copy(agg_sh.at[pl.ds(base, RPS)],
                    out_hbm.at[cid].at[pl.ds(base, RPS)])

  return _sc_gather_scatter_add


def kernel(x, edge_index, W_msg, b_msg, W_upd, b_upd):
  src = edge_index[0].astype(jnp.int32)
  dst = edge_index[1].astype(jnp.int32)
  pad = E_PAD - N_EDGES
  src_p = jnp.concatenate([src, jnp.zeros((pad,), jnp.int32)]).reshape(NW, NCH, K)
  dst_p = jnp.concatenate([dst, jnp.full((pad,), N_NODES, jnp.int32)]).reshape(NW, NCH, K)
  zero = jnp.zeros((RPS, D), jnp.float32)

  y = _msg_matmul(x, W_msg.T, b_msg.reshape(1, D))
  partials = _make_sc_gather_scatter_add()(y, src_p, dst_p, zero)
  return _update_matmul(partials, W_upd.T, b_upd.reshape(1, D))
